# baseline (speedup 1.0000x reference)
import functools

import jax
import jax.numpy as jnp
from jax import lax
from jax.experimental import pallas as pl
from jax.experimental.pallas import tpu as pltpu
from jax.experimental.pallas import tpu_sc as plsc

D = 64
CHUNK = 128
CB = 256
NB = 1954
H = NB * CB


def _pack_table(wt):
    d, v = wt.shape
    del v

    def body(lo_ref, hi_ref, o_ref):
        o_ref[...] = jnp.concatenate([lo_ref[...].T, hi_ref[...].T], axis=1)

    return pl.pallas_call(
        body,
        grid=(NB,),
        in_specs=[
            pl.BlockSpec((d, CB), lambda i: (0, i)),
            pl.BlockSpec((d, CB), lambda i: (0, jnp.minimum(i + NB, 3906))),
        ],
        out_specs=pl.BlockSpec((CB, 2 * d), lambda i: (i, 0)),
        out_shape=jax.ShapeDtypeStruct((H, 2 * d), jnp.float32),
    )(wt, wt)


def _make_gather(n_rows: int):
    info = plsc.get_sparse_core_info()
    nw = info.num_cores * info.num_subcores
    cpw = n_rows // (nw * CHUNK)

    mesh = plsc.VectorSubcoreMesh(core_axis_name="c", subcore_axis_name="s")

    @functools.partial(
        pl.kernel,
        mesh=mesh,
        out_type=jax.ShapeDtypeStruct((n_rows // 2, 2 * D), jnp.float32),
        scratch_types=[
            pltpu.VMEM((cpw, CHUNK), jnp.int32),
            pltpu.VMEM((2, CHUNK), jnp.int32),
            pltpu.VMEM((2, CHUNK), jnp.int32),
            pltpu.VMEM((2, CHUNK, 2 * D), jnp.float32),
            pltpu.VMEM((2, CHUNK // 2, 2 * D), jnp.float32),
            [pltpu.SemaphoreType.DMA] * 2,
            [pltpu.SemaphoreType.DMA] * 2,
        ],
        compiler_params=pltpu.CompilerParams(
            use_tc_tiling_on_sc=True, needs_layout_passes=False
        ),
    )
    def gather(idx_hbm, table_hbm, out_hbm, idx_v, q_v, hb_v, rows_v, wb_v, semg, semw):
        wid = lax.axis_index("s") * info.num_cores + lax.axis_index("c")
        base = wid * cpw
        pltpu.sync_copy(idx_hbm.at[wid], idx_v)
        iota = lax.iota(jnp.int32, 16)

        def fire_gather(j, slot):
            for g in range(CHUNK // 16):
                v = idx_v[j, pl.ds(g * 16, 16)]
                q_v[slot, pl.ds(g * 16, 16)] = jnp.where(v >= H, v - H, v)
            pltpu.async_copy(
                table_hbm.at[q_v.at[slot]], rows_v.at[slot], semg[slot]
            )

        def wait_gather(slot):
            pltpu.make_async_copy(
                out_hbm.at[pl.ds(0, CHUNK)], rows_v.at[slot], semg[slot]
            ).wait()

        def select(j, slot):
            for g in range(CHUNK // 16):
                v = idx_v[j, pl.ds(g * 16, 16)]
                hb_v[slot, pl.ds(g * 16, 16)] = jnp.where(v >= H, D, 0)

            def tok(i, carry):
                for k in range(8):
                    t = i * 8 + k
                    tsp = jnp.broadcast_to(t, (16,)).astype(jnp.int32)
                    h = plsc.load_gather(hb_v.at[slot], [tsp])
                    for c0 in range(0, D, 16):
                        vals = plsc.load_gather(
                            rows_v.at[slot], [tsp, h + c0 + iota]
                        )
                        wb_v[slot, t // 2, pl.ds((t % 2) * D + c0, 16)] = vals
                return carry

            lax.fori_loop(0, CHUNK // 8, tok, 0, unroll=2)

        def out_slice(g):
            start = pl.multiple_of((base + g) * (CHUNK // 2), CHUNK // 2)
            return out_hbm.at[pl.ds(start, CHUNK // 2)]

        def start_writeback(g, slot):
            pltpu.async_copy(wb_v.at[slot], out_slice(g), semw[slot])

        def wait_writeback(g, slot):
            pltpu.make_async_copy(wb_v.at[slot], out_slice(g), semw[slot]).wait()

        fire_gather(0, 0)
        fire_gather(1, 1)

        def step(i, carry):
            for k in range(2):
                g = 2 * i + k
                wait_gather(k)
                select(g, k)
                start_writeback(g, k)
            for k in range(2):
                g = 2 * i + k
                wait_writeback(g, k)
                fire_gather(g + 2, k)
            return carry

        lax.fori_loop(0, cpw // 2 - 1, step, 0)

        for k in range(2):
            g = cpw - 2 + k
            wait_gather(k)
            select(g, k)
            start_writeback(g, k)
        for k in range(2):
            wait_writeback(cpw - 2 + k, k)

    return gather


def kernel(token_ids, weight):
    b, s = token_ids.shape
    n_rows = b * s
    nw = 32
    cpw = n_rows // (nw * CHUNK)
    idx = token_ids.reshape(nw, cpw, CHUNK).astype(jnp.int32)
    table = _pack_table(weight.T)
    out = _make_gather(n_rows)(idx, table)
    return out.reshape(b, s, D)


# speedup vs baseline: 2.6138x; 2.6138x over previous
import functools

import jax
import jax.numpy as jnp
from jax import lax
from jax.experimental import pallas as pl
from jax.experimental.pallas import tpu as pltpu
from jax.experimental.pallas import tpu_sc as plsc

D = 64
CHUNK = 128
CB = 4096
NB = 123
H = NB * CB
LAST_BLOCK = 1000000 // CB


def _pack_table(wt):
    d, v = wt.shape
    del v

    def body(lo_ref, hi_ref, o_ref):
        o_ref[...] = jnp.concatenate([lo_ref[...].T, hi_ref[...].T], axis=1)

    return pl.pallas_call(
        body,
        grid=(NB,),
        in_specs=[
            pl.BlockSpec((d, CB), lambda i: (0, i)),
            pl.BlockSpec((d, CB), lambda i: (0, jnp.minimum(i + NB, LAST_BLOCK))),
        ],
        out_specs=pl.BlockSpec((CB, 2 * d), lambda i: (i, 0)),
        out_shape=jax.ShapeDtypeStruct((H, 2 * d), jnp.float32),
    )(wt, wt)


def _make_gather(n_rows: int):
    info = plsc.get_sparse_core_info()
    nw = info.num_cores * info.num_subcores
    cpw = n_rows // (nw * CHUNK)

    mesh = plsc.VectorSubcoreMesh(core_axis_name="c", subcore_axis_name="s")

    @functools.partial(
        pl.kernel,
        mesh=mesh,
        out_type=jax.ShapeDtypeStruct((n_rows // 2, 2 * D), jnp.float32),
        scratch_types=[
            pltpu.VMEM((cpw, CHUNK), jnp.int32),
            pltpu.VMEM((2, CHUNK), jnp.int32),
            pltpu.VMEM((2, CHUNK), jnp.int32),
            pltpu.VMEM((2, CHUNK, 2 * D), jnp.float32),
            pltpu.VMEM((2, CHUNK // 2, 2 * D), jnp.float32),
            [pltpu.SemaphoreType.DMA] * 2,
            [pltpu.SemaphoreType.DMA] * 2,
        ],
        compiler_params=pltpu.CompilerParams(
            use_tc_tiling_on_sc=True, needs_layout_passes=False
        ),
    )
    def gather(idx_hbm, table_hbm, out_hbm, idx_v, q_v, hb_v, rows_v, wb_v, semg, semw):
        wid = lax.axis_index("s") * info.num_cores + lax.axis_index("c")
        base = wid * cpw
        pltpu.sync_copy(idx_hbm.at[wid], idx_v)
        iota = lax.iota(jnp.int32, 16)

        def fire_gather(j, slot):
            for g in range(CHUNK // 16):
                v = idx_v[j, pl.ds(g * 16, 16)]
                q_v[slot, pl.ds(g * 16, 16)] = jnp.where(v >= H, v - H, v)
            pltpu.async_copy(
                table_hbm.at[q_v.at[slot]], rows_v.at[slot], semg[slot]
            )

        def wait_gather(slot):
            pltpu.make_async_copy(
                out_hbm.at[pl.ds(0, CHUNK)], rows_v.at[slot], semg[slot]
            ).wait()

        def select(j, slot):
            for g in range(CHUNK // 16):
                v = idx_v[j, pl.ds(g * 16, 16)]
                hb_v[slot, pl.ds(g * 16, 16)] = jnp.where(v >= H, D, 0)

            def tok(i, carry):
                for k in range(8):
                    t = i * 8 + k
                    tsp = jnp.broadcast_to(t, (16,)).astype(jnp.int32)
                    h = plsc.load_gather(hb_v.at[slot], [tsp])
                    for c0 in range(0, D, 16):
                        vals = plsc.load_gather(
                            rows_v.at[slot], [tsp, h + c0 + iota]
                        )
                        wb_v[slot, t // 2, pl.ds((t % 2) * D + c0, 16)] = vals
                return carry

            lax.fori_loop(0, CHUNK // 8, tok, 0, unroll=2)

        def out_slice(g):
            start = pl.multiple_of((base + g) * (CHUNK // 2), CHUNK // 2)
            return out_hbm.at[pl.ds(start, CHUNK // 2)]

        def start_writeback(g, slot):
            pltpu.async_copy(wb_v.at[slot], out_slice(g), semw[slot])

        def wait_writeback(g, slot):
            pltpu.make_async_copy(wb_v.at[slot], out_slice(g), semw[slot]).wait()

        fire_gather(0, 0)
        fire_gather(1, 1)

        def step(i, carry):
            for k in range(2):
                g = 2 * i + k
                wait_gather(k)
                select(g, k)
                start_writeback(g, k)
            for k in range(2):
                g = 2 * i + k
                wait_writeback(g, k)
                fire_gather(g + 2, k)
            return carry

        lax.fori_loop(0, cpw // 2 - 1, step, 0)

        for k in range(2):
            g = cpw - 2 + k
            wait_gather(k)
            select(g, k)
            start_writeback(g, k)
        for k in range(2):
            wait_writeback(cpw - 2 + k, k)

    return gather


def kernel(token_ids, weight):
    b, s = token_ids.shape
    n_rows = b * s
    nw = 32
    cpw = n_rows // (nw * CHUNK)
    idx = token_ids.reshape(nw, cpw, CHUNK).astype(jnp.int32)
    table = _pack_table(weight.T)
    out = _make_gather(n_rows)(idx, table)
    return out.reshape(b, s, D)


# speedup vs baseline: 2.6171x; 1.0013x over previous
import functools

import jax
import jax.numpy as jnp
from jax import lax
from jax.experimental import pallas as pl
from jax.experimental.pallas import tpu as pltpu
from jax.experimental.pallas import tpu_sc as plsc

D = 64
CHUNK = 128
CB = 4096
NB = 123
H = NB * CB
LAST_BLOCK = 1000000 // CB


def _pack_table(wt):
    d, v = wt.shape
    del v

    def body(lo_ref, hi_ref, o_ref):
        o_ref[...] = jnp.concatenate([lo_ref[...].T, hi_ref[...].T], axis=1)

    return pl.pallas_call(
        body,
        grid=(NB,),
        in_specs=[
            pl.BlockSpec((d, CB), lambda i: (0, i)),
            pl.BlockSpec((d, CB), lambda i: (0, jnp.minimum(i + NB, LAST_BLOCK))),
        ],
        out_specs=pl.BlockSpec((CB, 2 * d), lambda i: (i, 0)),
        out_shape=jax.ShapeDtypeStruct((H, 2 * d), jnp.float32),
    )(wt, wt)


def _make_gather(n_rows: int):
    info = plsc.get_sparse_core_info()
    nw = info.num_cores * info.num_subcores
    cpw = n_rows // (nw * CHUNK)

    mesh = plsc.VectorSubcoreMesh(core_axis_name="c", subcore_axis_name="s")

    @functools.partial(
        pl.kernel,
        mesh=mesh,
        out_type=jax.ShapeDtypeStruct((n_rows // 2, 2 * D), jnp.float32),
        scratch_types=[
            pltpu.VMEM((cpw, CHUNK), jnp.int32),
            pltpu.VMEM((2, CHUNK), jnp.int32),
            pltpu.VMEM((2, CHUNK), jnp.int32),
            pltpu.VMEM((2, CHUNK, 2 * D), jnp.float32),
            pltpu.VMEM((2, CHUNK // 2, 2 * D), jnp.float32),
            [pltpu.SemaphoreType.DMA] * 2,
            [pltpu.SemaphoreType.DMA] * 2,
        ],
        compiler_params=pltpu.CompilerParams(
            use_tc_tiling_on_sc=True, needs_layout_passes=False
        ),
    )
    def gather(idx_hbm, table_hbm, out_hbm, idx_v, q_v, hb_v, rows_v, wb_v, semg, semw):
        wid = lax.axis_index("s") * info.num_cores + lax.axis_index("c")
        base = wid * cpw
        pltpu.sync_copy(idx_hbm.at[wid], idx_v)
        iota = lax.iota(jnp.int32, 16)

        def fire_gather(j, slot):
            for g in range(CHUNK // 16):
                v = idx_v[j, pl.ds(g * 16, 16)]
                q_v[slot, pl.ds(g * 16, 16)] = jnp.where(v >= H, v - H, v)
            pltpu.async_copy(
                table_hbm.at[q_v.at[slot]], rows_v.at[slot], semg[slot]
            )

        def wait_gather(slot):
            pltpu.make_async_copy(
                out_hbm.at[pl.ds(0, CHUNK)], rows_v.at[slot], semg[slot]
            ).wait()

        def select(j, slot):
            for g in range(CHUNK // 16):
                v = idx_v[j, pl.ds(g * 16, 16)]
                hb_v[slot, pl.ds(g * 16, 16)] = jnp.where(v >= H, D, 0)

            def tok(i, carry):
                for k in range(8):
                    t = i * 8 + k
                    tsp = jnp.broadcast_to(t, (16,)).astype(jnp.int32)
                    h = plsc.load_gather(hb_v.at[slot], [tsp])
                    for c0 in range(0, D, 16):
                        vals = plsc.load_gather(
                            rows_v.at[slot], [tsp, h + c0 + iota]
                        )
                        wb_v[slot, t // 2, pl.ds((t % 2) * D + c0, 16)] = vals
                return carry

            lax.fori_loop(0, CHUNK // 8, tok, 0, unroll=4)

        def out_slice(g):
            start = pl.multiple_of((base + g) * (CHUNK // 2), CHUNK // 2)
            return out_hbm.at[pl.ds(start, CHUNK // 2)]

        def start_writeback(g, slot):
            pltpu.async_copy(wb_v.at[slot], out_slice(g), semw[slot])

        def wait_writeback(g, slot):
            pltpu.make_async_copy(wb_v.at[slot], out_slice(g), semw[slot]).wait()

        fire_gather(0, 0)
        fire_gather(1, 1)

        def step(i, carry):
            for k in range(2):
                g = 2 * i + k
                wait_gather(k)
                select(g, k)
                start_writeback(g, k)
            for k in range(2):
                g = 2 * i + k
                wait_writeback(g, k)
                fire_gather(g + 2, k)
            return carry

        lax.fori_loop(0, cpw // 2 - 1, step, 0)

        for k in range(2):
            g = cpw - 2 + k
            wait_gather(k)
            select(g, k)
            start_writeback(g, k)
        for k in range(2):
            wait_writeback(cpw - 2 + k, k)

    return gather


def kernel(token_ids, weight):
    b, s = token_ids.shape
    n_rows = b * s
    nw = 32
    cpw = n_rows // (nw * CHUNK)
    idx = token_ids.reshape(nw, cpw, CHUNK).astype(jnp.int32)
    table = _pack_table(weight.T)
    out = _make_gather(n_rows)(idx, table)
    return out.reshape(b, s, D)
